# baseline (speedup 1.0000x reference)
import functools

import jax
import jax.numpy as jnp
from jax import lax
from jax.experimental import pallas as pl
from jax.experimental.pallas import tpu as pltpu
from jax.experimental.pallas import tpu_sc as plsc

HIDDEN = 1024
B = 4 * 4096
DW = HIDDEN // 2
NC = 2
NS = 16
NW = NC * NS
BPW = B // NW
CH = 64
NCHUNK = BPW // CH

_mesh = plsc.VectorSubcoreMesh(core_axis_name="c", subcore_axis_name="s")


@functools.partial(
    pl.kernel,
    out_type=jax.ShapeDtypeStruct((B, DW), jnp.int32),
    mesh=_mesh,
    scratch_types=[
        pltpu.VMEM((BPW,), jnp.int32),
        pltpu.VMEM((CH, DW), jnp.int32),
        pltpu.VMEM((CH, DW), jnp.int32),
        pltpu.SemaphoreType.DMA,
        pltpu.SemaphoreType.DMA,
        pltpu.SemaphoreType.DMA,
        pltpu.SemaphoreType.DMA,
    ],
)
def _lookup(ids_hbm, table_hbm, out_hbm, idx_v, buf0, buf1, g0, g1, s0, s1):
    wid = lax.axis_index("s") * NC + lax.axis_index("c")
    base = wid * BPW
    pltpu.sync_copy(ids_hbm.at[pl.ds(base, BPW)], idx_v)

    bufs = (buf0, buf1)
    gsems = (g0, g1)
    ssems = (s0, s1)

    def start_gather(c):
        b = c % 2
        return pltpu.async_copy(
            table_hbm.at[idx_v.at[pl.ds(c * CH, CH)]], bufs[b], gsems[b])

    def start_out(c):
        b = c % 2
        return pltpu.async_copy(
            bufs[b], out_hbm.at[pl.ds(base + c * CH, CH)], ssems[b])

    gathers = [None] * NCHUNK
    outs = [None] * NCHUNK
    gathers[0] = start_gather(0)
    for c in range(NCHUNK):
        gathers[c].wait()
        outs[c] = start_out(c)
        if c + 1 < NCHUNK:
            if c >= 1:
                outs[c - 1].wait()
            gathers[c + 1] = start_gather(c + 1)
    outs[NCHUNK - 2].wait()
    outs[NCHUNK - 1].wait()


def kernel(token_type_ids, token_type_table):
    ids = jnp.reshape(token_type_ids, (B,)).astype(jnp.int32)
    table_w = lax.bitcast_convert_type(
        jnp.reshape(token_type_table, (2, DW, 2)), jnp.int32)
    out_w = _lookup(ids, table_w)
    out = lax.bitcast_convert_type(out_w, jnp.float16)
    return jnp.reshape(out, (B, HIDDEN))


# speedup vs baseline: 2.1346x; 2.1346x over previous
import functools

import jax
import jax.numpy as jnp
from jax import lax
from jax.experimental import pallas as pl
from jax.experimental.pallas import tpu as pltpu
from jax.experimental.pallas import tpu_sc as plsc

HIDDEN = 1024
B = 4 * 4096
DW = HIDDEN // 2
NC = 2
NS = 16
NW = NC * NS
BPW = B // NW
CH = 64
NCHUNK = BPW // CH
L = 16
HW = DW // 2
HV = HW // L

_mesh = plsc.VectorSubcoreMesh(core_axis_name="c", subcore_axis_name="s")


@functools.partial(
    pl.kernel,
    out_type=jax.ShapeDtypeStruct((B, DW), jnp.int32),
    mesh=_mesh,
    scratch_types=[
        pltpu.VMEM((BPW,), jnp.int32),
        pltpu.VMEM((2, DW), jnp.int32),
        pltpu.VMEM((CH, DW), jnp.int32),
        pltpu.VMEM((CH, DW), jnp.int32),
        pltpu.SemaphoreType.DMA,
        pltpu.SemaphoreType.DMA,
    ],
)
def _lookup(ids_hbm, table_hbm, out_hbm, idx_v, tab_v, buf0, buf1, s0, s1):
    wid = lax.axis_index("s") * NC + lax.axis_index("c")
    base = wid * BPW
    pltpu.sync_copy(ids_hbm.at[pl.ds(base, BPW)], idx_v)
    pltpu.sync_copy(table_hbm, tab_v)

    bufs = (buf0, buf1)
    ssems = (s0, s1)

    def build(buf, rows0):
        for h in range(2):
            x0 = [tab_v[0, pl.ds(h * HW + i * L, L)] for i in range(HV)]
            xr = [x0[i] ^ tab_v[1, pl.ds(h * HW + i * L, L)] for i in range(HV)]

            def body(r16, carry):
                idv = idx_v[pl.ds(rows0 + r16 * L, L)]
                for j in range(L):
                    m = jnp.broadcast_to(jnp.int32(0) - idv[j], (L,))
                    row = r16 * L + j
                    for i in range(HV):
                        buf[row, pl.ds(h * HW + i * L, L)] = x0[i] ^ (xr[i] & m)
                return carry

            lax.fori_loop(0, CH // L, body, 0)

    def wait_out(b):
        pltpu.make_async_copy(
            bufs[b], out_hbm.at[pl.ds(0, CH)], ssems[b]).wait()

    def pair_body(cp, carry):
        for b in range(2):
            rows0 = (2 * cp + b) * CH

            @pl.when(cp >= 1)
            def _():
                wait_out(b)

            build(bufs[b], rows0)
            pltpu.async_copy(
                bufs[b], out_hbm.at[pl.ds(base + rows0, CH)], ssems[b])
        return carry

    lax.fori_loop(0, NCHUNK // 2, pair_body, 0)
    wait_out(0)
    wait_out(1)


def kernel(token_type_ids, token_type_table):
    ids = jnp.reshape(token_type_ids, (B,)).astype(jnp.int32)
    table_w = lax.bitcast_convert_type(
        jnp.reshape(token_type_table, (2, DW, 2)), jnp.int32)
    out_w = _lookup(ids, table_w)
    out = lax.bitcast_convert_type(out_w, jnp.float16)
    return jnp.reshape(out, (B, HIDDEN))


# speedup vs baseline: 5.2025x; 2.4372x over previous
import functools

import jax
import jax.numpy as jnp
from jax import lax
from jax.experimental import pallas as pl
from jax.experimental.pallas import tpu as pltpu
from jax.experimental.pallas import tpu_sc as plsc

HIDDEN = 1024
B = 4 * 4096
NP = B // 2
NC = 2
NS = 16
NW = NC * NS
BPW = B // NW
PPW = BPW // 2
CH = 32
PPC = CH // 2
NCHUNK = BPW // CH
L = 16
NT = HIDDEN // L

_mesh = plsc.VectorSubcoreMesh(core_axis_name="c", subcore_axis_name="s")


@functools.partial(
    pl.kernel,
    out_type=jax.ShapeDtypeStruct((B, HIDDEN), jnp.float16),
    mesh=_mesh,
    compiler_params=pltpu.CompilerParams(use_tc_tiling_on_sc=True),
    scratch_types=[
        pltpu.VMEM((PPW,), jnp.int32),
        pltpu.VMEM((8, HIDDEN), jnp.float16),
        pltpu.VMEM((CH, HIDDEN), jnp.float16),
        pltpu.VMEM((CH, HIDDEN), jnp.float16),
        pltpu.SemaphoreType.DMA,
        pltpu.SemaphoreType.DMA,
    ],
)
def _lookup(combo_hbm, ptab_hbm, out_hbm, cv_v, pt_v, buf0, buf1, s0, s1):
    wid = lax.axis_index("s") * NC + lax.axis_index("c")
    base = wid * BPW
    pbase = wid * PPW
    pltpu.sync_copy(combo_hbm.at[pl.ds(pbase, PPW)], cv_v)
    pltpu.sync_copy(ptab_hbm, pt_v)

    bufs = (buf0, buf1)
    ssems = (s0, s1)

    def build(buf, c0):
        cv = cv_v[pl.ds(c0 * PPC, PPC)]
        for j in range(PPC):
            c2 = 2 * cv[j]
            for t in range(NT):
                buf[pl.ds(2 * j, 2), pl.ds(t * L, L)] = (
                    pt_v[pl.ds(c2, 2), pl.ds(t * L, L)])

    def wait_out(b):
        pltpu.make_async_copy(
            bufs[b], out_hbm.at[pl.ds(0, CH)], ssems[b]).wait()

    for b in range(2):
        def body(cp, carry, b=b):
            c0 = 2 * cp + b

            @pl.when(cp >= 1)
            def _():
                wait_out(b)

            build(bufs[b], c0)
            pltpu.async_copy(
                bufs[b], out_hbm.at[pl.ds(base + c0 * CH, CH)], ssems[b])
            return carry

        lax.fori_loop(0, NCHUNK // 2, body, 0)
    wait_out(0)
    wait_out(1)


def kernel(token_type_ids, token_type_table):
    ids = jnp.reshape(token_type_ids, (B,)).astype(jnp.int32)
    pairs = jnp.reshape(ids, (NP, 2))
    combo = pairs[:, 0] + 2 * pairs[:, 1]
    sel = jnp.array([0, 0, 1, 0, 0, 1, 1, 1], dtype=jnp.int32)
    ptab = token_type_table[sel, :]
    return _lookup(combo, ptab)
